# baseline (speedup 1.0000x reference)
import functools

import numpy as np

import jax
import jax.numpy as jnp
from jax.experimental import pallas as pl

_BF = jnp.bfloat16
_F32 = jnp.float32


def _np_constants(C, H, W, PS, RED, HID, E):
    HP, WP = H // PS, W // PS
    S = HP * WP
    l = np.arange(H * W)
    hh, ww = l // W, l % W
    PM = (((hh[:, None] // PS) * WP + (ww[:, None] // PS))
          == np.arange(S)[None, :]).astype(np.float32)
    si, sj = np.arange(S)[:, None] // WP, np.arange(S)[:, None] % WP
    oi, oj = np.arange(S)[None, :] // WP, np.arange(S)[None, :] % WP
    TT2 = np.zeros((S, 9 * S), np.float32)
    for ky in range(3):
        for kx in range(3):
            t = ky * 3 + kx
            TT2[:, t * S:(t + 1) * S] = ((si == oi + ky - 1) &
                                         (sj == oj + kx - 1))

    def gn_ind(nch, ngrp):
        return (np.arange(nch)[None, :] // (nch // ngrp)
                == np.arange(ngrp)[:, None]).astype(np.float32)

    G1 = gn_ind(C, 8)
    G2 = gn_ind(RED, 3)
    GE1 = gn_ind(HID, 8) / (HID // 8 * H * W)
    UE1 = gn_ind(HID, 8).T.copy()
    GE2 = gn_ind(C, 8) / (C // 8 * H * W)
    UE2 = gn_ind(C, 8).T.copy()
    return (PM, TT2, G1, G1.T.copy(), G2, G2.T.copy(), GE1, UE1, GE2, UE2)


def _split3(x):
    a = x.astype(_BF)
    t1 = x - a.astype(_F32)
    b = t1.astype(_BF)
    c = (t1 - b.astype(_F32)).astype(_BF)
    return a, b, c


def _d(a, b):
    return jnp.dot(a, b, preferred_element_type=_F32)


def _dot01(a01, x):
    h, m, l = _split3(x)
    return _d(a01, h) + _d(a01, m) + _d(a01, l)


def _dotw(w3, x):
    w1, w2, w3_ = w3
    h, m, l = _split3(x)
    return (_d(w1, h) + (_d(w1, m) + _d(w2, h))
            + (_d(w1, l) + _d(w2, m) + _d(w3_, h)))


def _silu(x):
    return x * jax.nn.sigmoid(x)


def _gn_fast(h, G, U, gamma, beta, scale, eps=1e-5):
    m = jnp.sum(_d(G, h), axis=1, keepdims=True)
    q = jnp.sum(_d(G, h * h), axis=1, keepdims=True)
    sc = jax.lax.rsqrt(q - m * m + eps)
    st = jnp.concatenate([sc, m * sc], axis=1)
    R = _d(U, st)
    a = R[:, 0:1] * gamma
    b = beta - R[:, 1:2] * gamma
    if scale is not None:
        a = a * scale
        b = b * scale
    return h * a + b


def _route_one(xb, S, C, RED, dw9_ref, g1_ref, b1_ref, g2_ref, b2_ref,
               pb_ref, pw1l, pw2l, pm_ref, tt2_ref, g1m_ref, u1m_ref,
               g2m_ref, u2m_ref):
    xh, xm, xl = _split3(xb)
    pm = pm_ref[...]
    p = (_d(xh, pm) + _d(xm, pm) + _d(xl, pm)) * (1.0 / 64.0)
    q9h, q9m, q9l = _split3(p)
    tt = tt2_ref[...]
    q9 = _d(q9h, tt) + _d(q9m, tt) + _d(q9l, tt)
    dw = dw9_ref[...]
    h = dw[:, 0:1] * q9[:, 0:S]
    for t in range(1, 9):
        h = h + dw[:, t:t + 1] * q9[:, t * S:(t + 1) * S]

    def gn(h, gref, uref, nelem, gamma, beta):
        m = jnp.sum(_dot01(gref[...], h), axis=1,
                    keepdims=True) * (1.0 / nelem)
        mr = _dot01(uref[...], m)
        d = h - mr
        v = jnp.sum(_dot01(gref[...], d * d), axis=1,
                    keepdims=True) * (1.0 / nelem)
        vr = _dot01(uref[...], v)
        return d / jnp.sqrt(vr + 1e-5) * gamma + beta

    h = _silu(gn(h, g1m_ref, u1m_ref, (C // 8) * S, g1_ref[...],
                 b1_ref[...]))
    h = _dotw(pw1l, h)
    h = _silu(gn(h, g2m_ref, u2m_ref, (RED // 3) * S, g2_ref[...],
                 b2_ref[...]))
    lm = _dotw(pw2l, h)
    logits = jnp.sum(lm, axis=1, keepdims=True) * (1.0 / S) + pb_ref[...]
    mx = jnp.max(logits)
    ex = jnp.exp(logits - mx)
    pr = ex / jnp.sum(ex)
    iot = jax.lax.broadcasted_iota(jnp.int32, pr.shape, 0)
    v1 = jnp.max(pr)
    i1 = jnp.min(jnp.where(pr >= v1, iot, 10000))
    pr2 = jnp.where(iot == i1, -1.0, pr)
    v2 = jnp.max(pr2)
    i2 = jnp.min(jnp.where(pr2 >= v2, iot, 10000))
    s = v1 + v2
    w1 = v1 / (s + 1e-9)
    w2 = v2 / (s + 1e-9)
    w1 = jnp.where(w1 > 0.01, w1, 0.0)
    w2 = jnp.where(w2 > 0.01, w2, 0.0)
    return i1, i2, w1, w2


def _moe_kernel(S, HID, C, RED, IMGS,
                x_ref, dw9_ref, g1_ref, b1_ref, g2_ref, b2_ref, pb_ref,
                pw1a_ref, pw1b_ref, pw1c_ref, pw2a_ref, pw2b_ref, pw2c_ref,
                ew1_ref, eg1_ref, eb1_ref, ew2_ref, eg2_ref, eb2_ref,
                pm_ref, tt2_ref, g1m_ref, u1m_ref, g2m_ref, u2m_ref,
                ge1_ref, ue1_ref, ge2_ref, ue2_ref, out_ref):
    pw1l = (pw1a_ref[...], pw1b_ref[...], pw1c_ref[...])
    pw2l = (pw2a_ref[...], pw2b_ref[...], pw2c_ref[...])
    for gi in range(IMGS):
        xb = x_ref[gi]
        e0, e1, w1, w2 = _route_one(
            xb, S, C, RED, dw9_ref, g1_ref, b1_ref, g2_ref, b2_ref,
            pb_ref, pw1l, pw2l, pm_ref, tt2_ref, g1m_ref, u1m_ref,
            g2m_ref, u2m_ref)
        xb16 = xb.astype(_BF)
        we = jnp.concatenate(
            [ew1_ref[pl.ds(e0, 1), :, :].reshape(HID, C),
             ew1_ref[pl.ds(e1, 1), :, :].reshape(HID, C)], axis=0)
        hd2 = _d(we, xb16)
        acc = None
        for k, (ei, wt) in enumerate(((e0, w1), (e1, w2))):
            hd = hd2[k * HID:(k + 1) * HID, :]
            eg1 = eg1_ref[pl.ds(ei, 1), :, :].reshape(HID, 1)
            eb1 = eb1_ref[pl.ds(ei, 1), :, :].reshape(HID, 1)
            hd = _silu(_gn_fast(hd, ge1_ref[...], ue1_ref[...],
                                eg1, eb1, None))
            we2 = ew2_ref[pl.ds(ei, 1), :, :].reshape(C, HID)
            od = _d(we2, hd.astype(_BF))
            eg2 = eg2_ref[pl.ds(ei, 1), :, :].reshape(C, 1)
            eb2 = eb2_ref[pl.ds(ei, 1), :, :].reshape(C, 1)
            od = _gn_fast(od, ge2_ref[...], ue2_ref[...], eg2, eb2, wt)
            acc = od if acc is None else acc + od
        out_ref[gi] = acc


def kernel(x, r_dw, r_gn1_g, r_gn1_b, r_pw1, r_gn2_g, r_gn2_b,
           r_pw2_w, r_pw2_b, e_w1, e_gn1_g, e_gn1_b, e_w2, e_gn2_g, e_gn2_b):
    B, C, H, W = x.shape
    E, HID = e_w1.shape[0], e_w1.shape[1]
    RED = r_pw1.shape[0]
    PS = 8
    S = (H // PS) * (W // PS)
    HW = H * W

    (PM, TT2, G1, U1, G2, U2, GE1, UE1, GE2,
     UE2) = _np_constants(C, H, W, PS, RED, HID, E)

    def split3_out(w):
        a = w.astype(_BF)
        t1 = w - a.astype(_F32)
        b = t1.astype(_BF)
        c = (t1 - b.astype(_F32)).astype(_BF)
        return a, b, c

    x_r = x.reshape(B, C, HW)
    pw1l = split3_out(r_pw1.reshape(RED, C))
    pw2l = split3_out(r_pw2_w.reshape(E, RED))
    bfc = lambda a: jnp.asarray(a, dtype=_BF)
    ins = (x_r, r_dw.reshape(C, 9),
           r_gn1_g.reshape(C, 1), r_gn1_b.reshape(C, 1),
           r_gn2_g.reshape(RED, 1), r_gn2_b.reshape(RED, 1),
           r_pw2_b.reshape(E, 1),
           pw1l[0], pw1l[1], pw1l[2], pw2l[0], pw2l[1], pw2l[2],
           e_w1.reshape(E, HID, C).astype(_BF),
           e_gn1_g.reshape(E, HID, 1), e_gn1_b.reshape(E, HID, 1),
           e_w2.reshape(E, C, HID).astype(_BF),
           e_gn2_g.reshape(E, C, 1), e_gn2_b.reshape(E, C, 1),
           bfc(PM), bfc(TT2), bfc(G1), bfc(U1), bfc(G2), bfc(U2),
           jnp.asarray(GE1), jnp.asarray(UE1),
           jnp.asarray(GE2), jnp.asarray(UE2))

    def full_spec(a):
        nd = a.ndim
        return pl.BlockSpec(a.shape, lambda b, _n=nd: (0,) * _n)

    IMGS = 2
    in_specs = [pl.BlockSpec((IMGS, C, HW), lambda b: (b, 0, 0))]
    in_specs += [full_spec(a) for a in ins[1:]]
    body = functools.partial(_moe_kernel, S, HID, C, RED, IMGS)
    out = pl.pallas_call(
        body,
        grid=(B // IMGS,),
        in_specs=in_specs,
        out_specs=pl.BlockSpec((IMGS, C, HW), lambda b: (b, 0, 0)),
        out_shape=jax.ShapeDtypeStruct((B, C, HW), _F32),
    )(*ins)
    return out.reshape(B, C, H, W)


# speedup vs baseline: 1.4012x; 1.4012x over previous
import functools

import numpy as np

import jax
import jax.numpy as jnp
from jax.experimental import pallas as pl
from jax.experimental.pallas import tpu as pltpu

_BF = jnp.bfloat16
_F32 = jnp.float32


def _np_constants(B, C, H, W, PS, RED, HID, E):
    HP, WP = H // PS, W // PS
    S = HP * WP
    l = np.arange(H * W)
    hh, ww = l // W, l % W
    PM = (((hh[:, None] // PS) * WP + (ww[:, None] // PS))
          == np.arange(S)[None, :]).astype(np.float32)
    si, sj = np.arange(S)[:, None] // WP, np.arange(S)[:, None] % WP
    oi, oj = np.arange(S)[None, :] // WP, np.arange(S)[None, :] % WP
    TT2 = np.zeros((S, 9 * S), np.float32)
    for ky in range(3):
        for kx in range(3):
            t = ky * 3 + kx
            TT2[:, t * S:(t + 1) * S] = ((si == oi + ky - 1) &
                                         (sj == oj + kx - 1))

    def gn_ind(nch, ngrp):
        return (np.arange(nch)[None, :] // (nch // ngrp)
                == np.arange(ngrp)[:, None]).astype(np.float32)

    def bkron(a):
        r, c = a.shape
        out = np.zeros((B * r, B * c), a.dtype)
        for b in range(B):
            out[b * r:(b + 1) * r, b * c:(b + 1) * c] = a
        return out

    G1 = gn_ind(C, 8)
    G2 = gn_ind(RED, 3)
    G1B, U1B = bkron(G1), bkron(G1.T.copy())
    G2B, U2B = bkron(G2), bkron(G2.T.copy())
    GE1 = gn_ind(HID, 8) / (HID // 8 * H * W)
    UE1 = gn_ind(HID, 8).T.copy()
    GE2 = gn_ind(C, 8) / (C // 8 * H * W)
    UE2 = gn_ind(C, 8).T.copy()
    IND = (np.arange(B * E)[:, None] // E
           == np.arange(B)[None, :]).astype(np.float32)
    SEL = (np.arange(B * E)[None, :] % E
           == np.arange(E)[:, None]).astype(np.float32)
    return PM, TT2, G1B, U1B, G2B, U2B, GE1, UE1, GE2, UE2, IND, SEL


def _split3(x):
    a = x.astype(_BF)
    t1 = x - a.astype(_F32)
    b = t1.astype(_BF)
    c = (t1 - b.astype(_F32)).astype(_BF)
    return a, b, c


def _split3_out(w):
    a = w.astype(_BF)
    t1 = w - a.astype(_F32)
    b = t1.astype(_BF)
    c = (t1 - b.astype(_F32)).astype(_BF)
    return a, b, c


def _d(a, b):
    return jnp.dot(a, b, preferred_element_type=_F32)


def _dot01(a01, x):
    h, m, l = _split3(x)
    return _d(a01, h) + _d(a01, m) + _d(a01, l)


def _dot01_r(x, a01):
    h, m, l = _split3(x)
    return _d(h, a01) + _d(m, a01) + _d(l, a01)


def _dotw(w3, x):
    w1, w2, w3_ = w3
    h, m, l = _split3(x)
    return (_d(w1, h) + (_d(w1, m) + _d(w2, h))
            + (_d(w1, l) + _d(w2, m) + _d(w3_, h)))


def _silu(x):
    return x * jax.nn.sigmoid(x)


def _router_kernel(BC, HW, S, E, B, C, RED,
                   x_ref, dw9t_ref, g1_ref, b1_ref, g2_ref, b2_ref, pb_ref,
                   pw1a_ref, pw1b_ref, pw1c_ref, pw2a_ref, pw2b_ref,
                   pw2c_ref, pm_ref, tt2_ref, g1b_ref, u1b_ref, g2b_ref,
                   u2b_ref, ind_ref, sel_ref, out_ref):
    x2 = x_ref[...].reshape(BC, HW)
    p = _dot01_r(x2, pm_ref[...]) * (1.0 / (HW // S))
    q9 = _dot01_r(p, tt2_ref[...])
    dw = dw9t_ref[...]
    h = dw[:, 0:1] * q9[:, 0:S]
    for t in range(1, 9):
        h = h + dw[:, t:t + 1] * q9[:, t * S:(t + 1) * S]

    def gn(h, gref, uref, nelem, gamma, beta):
        m = jnp.sum(_dot01(gref[...], h), axis=1,
                    keepdims=True) * (1.0 / nelem)
        mr = _dot01(uref[...], m)
        d = h - mr
        v = jnp.sum(_dot01(gref[...], d * d), axis=1,
                    keepdims=True) * (1.0 / nelem)
        vr = _dot01(uref[...], v)
        return d / jnp.sqrt(vr + 1e-5) * gamma + beta

    h = _silu(gn(h, g1b_ref, u1b_ref, (C // 8) * S,
                 g1_ref[...], b1_ref[...]))
    h = _dotw((pw1a_ref[...], pw1b_ref[...], pw1c_ref[...]), h)
    h = _silu(gn(h, g2b_ref, u2b_ref, (RED // 3) * S,
                 g2_ref[...], b2_ref[...]))
    lm = _dotw((pw2a_ref[...], pw2b_ref[...], pw2c_ref[...]), h)
    logits = jnp.sum(lm, axis=1, keepdims=True) * (1.0 / S) + pb_ref[...]
    L = _dot01(sel_ref[...], logits * ind_ref[...])
    mx = jnp.max(L, axis=0, keepdims=True)
    ex = jnp.exp(L - mx)
    pr = ex / jnp.sum(ex, axis=0, keepdims=True)
    iot = jax.lax.broadcasted_iota(jnp.int32, pr.shape, 0)
    v1 = jnp.max(pr, axis=0, keepdims=True)
    i1 = jnp.min(jnp.where(pr >= v1, iot, 10000), axis=0, keepdims=True)
    pr2 = jnp.where(iot == i1, -1.0, pr)
    v2 = jnp.max(pr2, axis=0, keepdims=True)
    i2 = jnp.min(jnp.where(pr2 >= v2, iot, 10000), axis=0, keepdims=True)
    s = v1 + v2
    w1 = v1 / (s + 1e-9)
    w2 = v2 / (s + 1e-9)
    w1 = jnp.where(w1 > 0.01, w1, 0.0)
    w2 = jnp.where(w2 > 0.01, w2, 0.0)
    zeros = jnp.zeros((4, B), _F32)
    out_ref[...] = jnp.concatenate(
        [i1.astype(_F32), i2.astype(_F32), w1, w2, zeros], axis=0)


def _gn_fast(h, G, U, gamma, beta, scale, eps=1e-5):
    m = jnp.sum(_d(G, h), axis=1, keepdims=True)
    q = jnp.sum(_d(G, h * h), axis=1, keepdims=True)
    sc = jax.lax.rsqrt(q - m * m + eps)
    st = jnp.concatenate([sc, m * sc], axis=1)
    R = _d(U, st)
    a = R[:, 0:1] * gamma
    b = beta - R[:, 1:2] * gamma
    if scale is not None:
        a = a * scale
        b = b * scale
    return h * a + b


def _expert_kernel(HID, C, IMGS,
                   idx_ref, wts_ref, x_ref, ew1_ref, eg1_ref, eb1_ref,
                   ew2_ref, eg2_ref, eb2_ref, ge1_ref, ue1_ref, ge2_ref,
                   ue2_ref, out_ref):
    b = pl.program_id(0)
    for gi in range(IMGS):
        xb16 = x_ref[gi].astype(_BF)
        base = (b * IMGS + gi) * 2
        e0 = idx_ref[base]
        e1 = idx_ref[base + 1]
        we = jnp.concatenate(
            [ew1_ref[pl.ds(e0, 1), :, :].reshape(HID, C),
             ew1_ref[pl.ds(e1, 1), :, :].reshape(HID, C)], axis=0)
        hd2 = _d(we, xb16)
        acc = None
        for k, ei in ((0, e0), (1, e1)):
            wt = wts_ref[base + k]
            hd = hd2[k * HID:(k + 1) * HID, :]
            eg1 = eg1_ref[pl.ds(ei, 1), :, :].reshape(HID, 1)
            eb1 = eb1_ref[pl.ds(ei, 1), :, :].reshape(HID, 1)
            hd = _silu(_gn_fast(hd, ge1_ref[...], ue1_ref[...],
                                eg1, eb1, None))
            we2 = ew2_ref[pl.ds(ei, 1), :, :].reshape(C, HID)
            od = _d(we2, hd.astype(_BF))
            eg2 = eg2_ref[pl.ds(ei, 1), :, :].reshape(C, 1)
            eb2 = eb2_ref[pl.ds(ei, 1), :, :].reshape(C, 1)
            od = _gn_fast(od, ge2_ref[...], ue2_ref[...], eg2, eb2, wt)
            acc = od if acc is None else acc + od
        out_ref[gi] = acc


def kernel(x, r_dw, r_gn1_g, r_gn1_b, r_pw1, r_gn2_g, r_gn2_b,
           r_pw2_w, r_pw2_b, e_w1, e_gn1_g, e_gn1_b, e_w2, e_gn2_g, e_gn2_b):
    B, C, H, W = x.shape
    E, HID = e_w1.shape[0], e_w1.shape[1]
    RED = r_pw1.shape[0]
    PS = 8
    S = (H // PS) * (W // PS)
    HW = H * W

    (PM, TT2, G1B, U1B, G2B, U2B, GE1, UE1, GE2, UE2, IND,
     SEL) = _np_constants(B, C, H, W, PS, RED, HID, E)

    x_r = x.reshape(B, C, HW)
    dw9t = jnp.tile(r_dw.reshape(C, 9), (B, 1))
    pw1k = jnp.kron(jnp.eye(B, dtype=x.dtype), r_pw1.reshape(RED, C))
    pw2k = jnp.kron(jnp.eye(B, dtype=x.dtype), r_pw2_w.reshape(E, RED))
    pw1l = _split3_out(pw1k)
    pw2l = _split3_out(pw2k)
    g1t = jnp.tile(r_gn1_g.reshape(C, 1), (B, 1))
    b1t = jnp.tile(r_gn1_b.reshape(C, 1), (B, 1))
    g2t = jnp.tile(r_gn2_g.reshape(RED, 1), (B, 1))
    b2t = jnp.tile(r_gn2_b.reshape(RED, 1), (B, 1))
    pbt = jnp.tile(r_pw2_b.reshape(E, 1), (B, 1))

    bfc = lambda a: jnp.asarray(a, dtype=_BF)
    r_ins = (x_r, dw9t, g1t, b1t, g2t, b2t, pbt,
             pw1l[0], pw1l[1], pw1l[2], pw2l[0], pw2l[1], pw2l[2],
             bfc(PM), bfc(TT2), bfc(G1B), bfc(U1B), bfc(G2B), bfc(U2B),
             jnp.asarray(IND), bfc(SEL))

    def full_spec(a):
        nd = a.ndim
        return pl.BlockSpec(a.shape, lambda *_, _n=nd: (0,) * _n)

    rbody = functools.partial(_router_kernel, B * C, HW, S, E, B, C, RED)
    rout = pl.pallas_call(
        rbody,
        grid=(1,),
        in_specs=[full_spec(a) for a in r_ins],
        out_specs=pl.BlockSpec((8, B), lambda i: (0, 0)),
        out_shape=jax.ShapeDtypeStruct((8, B), _F32),
    )(*r_ins)

    idx_flat = jnp.stack([rout[0], rout[1]], axis=1).reshape(-1)
    idx_flat = idx_flat.astype(jnp.int32)
    wts_flat = jnp.stack([rout[2], rout[3]], axis=1).reshape(-1)

    IMGS = 8
    e_ins = (x_r,
             e_w1.reshape(E, HID, C).astype(_BF),
             e_gn1_g.reshape(E, HID, 1), e_gn1_b.reshape(E, HID, 1),
             e_w2.reshape(E, C, HID).astype(_BF),
             e_gn2_g.reshape(E, C, 1), e_gn2_b.reshape(E, C, 1),
             jnp.asarray(GE1), jnp.asarray(UE1),
             jnp.asarray(GE2), jnp.asarray(UE2))
    in_specs = [pl.BlockSpec((IMGS, C, HW), lambda b, *_: (b, 0, 0))]
    in_specs += [full_spec(a) for a in e_ins[1:]]
    ebody = functools.partial(_expert_kernel, HID, C, IMGS)
    out = pl.pallas_call(
        ebody,
        grid_spec=pltpu.PrefetchScalarGridSpec(
            num_scalar_prefetch=2,
            grid=(B // IMGS,),
            in_specs=in_specs,
            out_specs=pl.BlockSpec((IMGS, C, HW), lambda b, *_: (b, 0, 0)),
        ),
        out_shape=jax.ShapeDtypeStruct((B, C, HW), _F32),
    )(idx_flat, wts_flat, *e_ins)
    return out.reshape(B, C, H, W)
